# Initial kernel scaffold; baseline (speedup 1.0000x reference)
import jax
import jax.numpy as jnp
from jax.experimental import pallas as pl


def kernel(span_vecs, span_mask, span_begin, span_end, sequence_lengths, W0, b0, W1, b1):
    raise NotImplementedError("write your pallas kernel here")


# speedup vs baseline: 1.5434x; 1.5434x over previous
import functools

import jax
import jax.numpy as jnp
from jax import lax
from jax.experimental import pallas as pl
from jax.experimental.pallas import tpu as pltpu
from jax.experimental.pallas import tpu_sc as plsc

_LANES = 128
_BLK = 1920


def _score_body(x_ref, m_ref, w0_ref, b0_ref, w1_ref, b1_ref, out_ref):
    h = jnp.maximum(
        jnp.dot(x_ref[...], w0_ref[...], preferred_element_type=jnp.float32)
        + b0_ref[...],
        0.0,
    )
    s = jnp.dot(h, w1_ref[...], preferred_element_type=jnp.float32) + b1_ref[...]
    out_ref[...] = s - (1.0 - m_ref[...]) * 10000.0


def _topk_body(k, rows, s_ref, bg_ref, en_ref,
               idx_ref, fs_ref, fb_ref, fe_ref, sq_ref, tr_ref):
    s = s_ref[...]
    bits = lax.bitcast_convert_type(s, jnp.int32)
    ikey = bits ^ (lax.shift_right_arithmetic(bits, 31) & jnp.int32(0x7FFFFFFF))

    ki = jnp.int32(k)
    kf = jnp.float32(k)

    cnt0 = jnp.sum((ikey >= 0).astype(jnp.int32))
    pos_half = cnt0 >= ki
    lo0 = jnp.where(pos_half, jnp.int32(0), jnp.int32(-2147483648))
    hi0 = jnp.where(pos_half, jnp.int32(2147483647), jnp.int32(-1))

    def bs_body(_, carry):
        lo, hi = carry
        d = hi - lo
        mid = lo + lax.shift_right_arithmetic(d, 1) + (d & 1)
        cnt = jnp.sum((ikey >= mid).astype(jnp.int32))
        ge = cnt >= ki
        return (jnp.where(ge, mid, lo), jnp.where(ge, hi, mid - 1))

    t_key, _ = lax.fori_loop(0, 31, bs_body, (lo0, hi0))

    gt = ikey > t_key
    eq = ikey == t_key
    eqf = eq.astype(jnp.float32)
    need = kf - jnp.sum(gt.astype(jnp.float32))

    ii_l = lax.broadcasted_iota(jnp.int32, (_LANES, _LANES), 0)
    jj_l = lax.broadcasted_iota(jnp.int32, (_LANES, _LANES), 1)
    upper = (ii_l <= jj_l).astype(jnp.float32)
    ii_r = lax.broadcasted_iota(jnp.int32, (rows, rows), 0)
    jj_r = lax.broadcasted_iota(jnp.int32, (rows, rows), 1)
    strict_lower = (jj_r < ii_r).astype(jnp.float32)

    hi_p = lax.Precision.HIGHEST

    def rm_incl(x):
        rowincl = jnp.dot(x, upper, preferred_element_type=jnp.float32,
                          precision=hi_p)
        rowtot = rowincl[:, _LANES - 1:_LANES]
        rowoff = jnp.dot(strict_lower, rowtot, preferred_element_type=jnp.float32,
                         precision=hi_p)
        return rowincl + rowoff, rowoff + rowtot

    eq_incl, _ = rm_incl(eqf)
    eq_excl = eq_incl - eqf
    sel = jnp.logical_or(gt, jnp.logical_and(eq, eq_excl < need))
    s_incl, rowend = rm_incl(sel.astype(jnp.float32))

    kpad = idx_ref.shape[1]
    t_row = lax.broadcasted_iota(jnp.int32, (1, kpad), 1).astype(jnp.float32) + 1.0
    cmp = (rowend < t_row).astype(jnp.float32)
    r_row = jnp.dot(jnp.ones((1, rows), jnp.float32), cmp,
                    preferred_element_type=jnp.float32, precision=hi_p)
    iota_r = lax.broadcasted_iota(jnp.int32, (rows, kpad), 0).astype(jnp.float32)
    onehot_rt = (iota_r == r_row).astype(jnp.float32)
    dn = (((0,), (0,)), ((), ()))
    rowvals = lax.dot_general(s_incl, onehot_rt, dn,
                              preferred_element_type=jnp.float32,
                              precision=hi_p)
    pos_row = jnp.sum((rowvals < t_row).astype(jnp.float32), axis=0,
                      keepdims=True)
    valid = t_row <= kf
    flat = jnp.where(valid, r_row * float(_LANES) + pos_row, 0.0)
    idx_ref[...] = flat.astype(jnp.int32)

    iota_l = lax.broadcasted_iota(jnp.int32, (_LANES, kpad), 0).astype(jnp.float32)
    onehot_lt = (iota_l == pos_row).astype(jnp.float32)

    def gath(v2d):
        vrow = lax.dot_general(v2d, onehot_rt, dn,
                               preferred_element_type=jnp.float32,
                               precision=hi_p)
        return jnp.sum(vrow * onehot_lt, axis=0, keepdims=True)

    fs_ref[...] = jnp.where(valid, gath(s), 0.0)
    fb_ref[...] = jnp.where(valid, gath(bg_ref[...]), 0.0)
    fe_ref[...] = jnp.where(valid, gath(en_ref[...]), 0.0)

    sq_ref[...] = jnp.ones((k, k), jnp.float32)
    ii_k = lax.broadcasted_iota(jnp.int32, (k, k), 0)
    jj_k = lax.broadcasted_iota(jnp.int32, (k, k), 1)
    tr_ref[...] = (jj_k <= ii_k).astype(jnp.float32)


def _sc_gather(table, idx, d, kg):
    nc, ns = 2, 16
    nw = nc * ns
    b_per_w = kg // nw
    mesh = plsc.VectorSubcoreMesh(core_axis_name="c", subcore_axis_name="s")

    @functools.partial(
        pl.kernel,
        mesh=mesh,
        out_type=jax.ShapeDtypeStruct((kg, d), jnp.float32),
        scratch_types=[
            pltpu.VMEM((b_per_w,), jnp.int32),
            pltpu.VMEM((b_per_w, d), jnp.float32),
            pltpu.SemaphoreType.DMA,
        ],
    )
    def gather_k(table_hbm, idx_hbm, out_hbm, idx_v, rows_v, sem):
        wid = lax.axis_index("s") * nc + lax.axis_index("c")
        base = wid * b_per_w
        pltpu.sync_copy(idx_hbm.at[pl.ds(base, b_per_w)], idx_v)
        pltpu.async_copy(table_hbm.at[idx_v], rows_v, sem).wait()
        pltpu.sync_copy(rows_v, out_hbm.at[pl.ds(base, b_per_w)])

    return gather_k(table, idx)


def kernel(span_vecs, span_mask, span_begin, span_end, sequence_lengths,
           W0, b0, W1, b1):
    b, s, w, d = span_vecs.shape
    h = W0.shape[1]
    n = s * w
    k = min(3900, int(0.4 * s))
    rows = n // _LANES
    kpad = 1024

    x = span_vecs.reshape(n, d)
    m = span_mask.reshape(n, 1)

    prune_flat = pl.pallas_call(
        _score_body,
        grid=(n // _BLK,),
        in_specs=[
            pl.BlockSpec((_BLK, d), lambda i: (i, 0)),
            pl.BlockSpec((_BLK, 1), lambda i: (i, 0)),
            pl.BlockSpec((d, h), lambda i: (0, 0)),
            pl.BlockSpec((1, h), lambda i: (0, 0)),
            pl.BlockSpec((h, 1), lambda i: (0, 0)),
            pl.BlockSpec((1, 1), lambda i: (0, 0)),
        ],
        out_specs=pl.BlockSpec((_BLK, 1), lambda i: (i, 0)),
        out_shape=jax.ShapeDtypeStruct((n, 1), jnp.float32),
    )(x, m, W0, b0.reshape(1, h), W1, b1.reshape(1, 1))

    scores2d = prune_flat.reshape(rows, _LANES)
    bg2d = span_begin.reshape(rows, _LANES)
    en2d = span_end.reshape(rows, _LANES)

    idx_row, fs_row, fb_row, fe_row, sq, tr = pl.pallas_call(
        functools.partial(_topk_body, k, rows),
        out_shape=(
            jax.ShapeDtypeStruct((1, kpad), jnp.int32),
            jax.ShapeDtypeStruct((1, kpad), jnp.float32),
            jax.ShapeDtypeStruct((1, kpad), jnp.float32),
            jax.ShapeDtypeStruct((1, kpad), jnp.float32),
            jax.ShapeDtypeStruct((k, k), jnp.float32),
            jax.ShapeDtypeStruct((k, k), jnp.float32),
        ),
    )(scores2d, bg2d, en2d)

    filt_vecs = _sc_gather(x, idx_row.reshape(kpad), d, kpad)[:k][None]

    prune_scores = prune_flat.reshape(b, s, w, 1)
    sorted_idx = idx_row[:, :k]
    filt_scores = fs_row[:, :k][..., None]
    filt_begin = fb_row[:, :k][..., None]
    filt_end = fe_row[:, :k][..., None]

    return (prune_scores, filt_vecs, filt_scores, filt_begin, filt_end,
            sorted_idx, sq[None], tr[None])
